# baseline (speedup 1.0000x reference)
import jax
import jax.numpy as jnp
from jax import lax
from jax.experimental import pallas as pl
from jax.experimental.pallas import tpu as pltpu

N = 4096
HID = 256
TOP_K = 10
BLK = 512
NBLK = N // BLK
SENT = -1e9


def _body(x0, x1, x2, w01, w02, w11, w12, w21, w22, aw, fw, fb,
          h_out, w_out, nrm_s, nt_s, rsum_s, et_s, keep_s):
    i = pl.program_id(0)

    @pl.when(i < NBLK)
    def _phase_a():
        a = aw[...]
        a = a - jnp.max(a, axis=1, keepdims=True)
        e = jnp.exp(a)
        a = e / jnp.sum(e, axis=1, keepdims=True)

        def mlp(x, w1, w2):
            h = jnp.maximum(jnp.dot(x[...], w1[...], preferred_element_type=jnp.float32), 0.0)
            return jnp.dot(h, w2[...], preferred_element_type=jnp.float32)

        fsum = (a[0, 0] * mlp(x0, w01, w02) + a[0, 1] * mlp(x1, w11, w12)
                + a[0, 2] * mlp(x2, w21, w22))
        fused = jnp.dot(fsum, fw[...], preferred_element_type=jnp.float32) + fb[...]
        nrm = jnp.sqrt(jnp.sum(fused * fused, axis=1, keepdims=True))
        nrm = jnp.maximum(nrm, 1e-12)
        normed = fused / nrm
        nrm_s[pl.ds(i * BLK, BLK), :] = normed
        nt_s[:, pl.ds(i, 1), :] = normed.T.reshape(HID, 1, BLK)
        rsum_s[pl.ds(i * BLK, BLK), :] = jnp.sum(fused, axis=1, keepdims=True)

    @pl.when(jnp.logical_and(i >= NBLK, i < 2 * NBLK))
    def _phase_b():
        j = i - NBLK
        r0 = j * BLK
        rids_i = r0 + lax.broadcasted_iota(jnp.int32, (BLK, 1), 0)
        rids = rids_i.astype(jnp.float32)
        cols = lax.broadcasted_iota(jnp.int32, (BLK, N), 1).astype(jnp.float32)
        nb = nrm_s[pl.ds(r0, BLK), :]
        nt = nt_s[...].reshape(HID, N)
        sim = jnp.dot(nb, nt, preferred_element_type=jnp.float32)
        sim = jnp.where(cols == rids, SENT, sim)
        picks = [rids]
        for _ in range(TOP_K):
            m = jnp.max(sim, axis=1, keepdims=True)
            cand = jnp.where(sim == m, cols, float(N))
            jj = jnp.min(cand, axis=1, keepdims=True)
            sim = jnp.where(cols == jj, SENT, sim)
            picks.append(jj)
        selmask = (sim == SENT).astype(jnp.float32)
        msum = jnp.dot(selmask, rsum_s[...], preferred_element_type=jnp.float32)
        w = jax.nn.sigmoid(msum / float((TOP_K + 1) * HID))
        keep = w > 0.0
        picks += [jnp.full((BLK, 1), -1.0, jnp.float32)] * (16 - len(picks))
        edges = jnp.concatenate(picks, axis=1).astype(jnp.int32)
        et_s[:, pl.ds(j, 1), :] = edges.T.reshape(16, 1, BLK)
        keep_s[pl.ds(j, 1), :] = jnp.where(keep, 1.0, 0.0).T.reshape(1, BLK)
        w_out[...] = jnp.where(keep, w, 0.0).T.reshape(1, BLK)

    @pl.when(i >= 2 * NBLK)
    def _phase_c():
        j = i - 2 * NBLK
        r0 = j * BLK
        rids = (r0 + lax.broadcasted_iota(jnp.int32, (BLK, 1), 0)).astype(jnp.float32)
        for g in range(NBLK):
            et_g = et_s[:, g, :].astype(jnp.float32)
            kf = keep_s[g:g + 1, :]
            acc = et_g[0:1, :] == rids
            for t in range(1, TOP_K + 1):
                acc = jnp.logical_or(acc, et_g[t:t + 1, :] == rids)
            h_out[:, g * BLK:(g + 1) * BLK] = jnp.where(acc, kf, 0.0)


def kernel(x0, x1, x2, mW0_1, mb0_1, mW0_2, mb0_2, mW1_1, mb1_1, mW1_2, mb1_2,
           mW2_1, mb2_1, mW2_2, mb2_2, attn_weights, fW, fb):
    f32 = jnp.float32
    aw2 = attn_weights.reshape(1, 3)
    fb2 = fb.reshape(1, HID)

    whole = lambda shape: pl.BlockSpec(shape, lambda i: tuple(0 for _ in shape))
    rows = lambda w: pl.BlockSpec((BLK, w), lambda i: (jnp.minimum(i, NBLK - 1), 0))

    Hmat, w = pl.pallas_call(
        _body,
        grid=(3 * NBLK,),
        in_specs=[
            rows(256), rows(512), rows(128),
            whole((256, HID)), whole((HID, HID)),
            whole((512, HID)), whole((HID, HID)),
            whole((128, HID)), whole((HID, HID)),
            whole((1, 3)), whole((HID, HID)), whole((1, HID)),
        ],
        out_specs=[
            pl.BlockSpec((BLK, N), lambda i: (jnp.clip(i - 2 * NBLK, 0, NBLK - 1), 0)),
            pl.BlockSpec((1, BLK), lambda i: (0, jnp.clip(i - NBLK, 0, NBLK - 1))),
        ],
        out_shape=[
            jax.ShapeDtypeStruct((N, N), f32),
            jax.ShapeDtypeStruct((1, N), f32),
        ],
        scratch_shapes=[
            pltpu.VMEM((N, HID), f32),
            pltpu.VMEM((HID, NBLK, BLK), f32),
            pltpu.VMEM((N, 1), f32),
            pltpu.VMEM((16, NBLK, BLK), jnp.int32),
            pltpu.VMEM((NBLK, BLK), f32),
        ],
    )(x0, x1, x2, mW0_1, mW0_2, mW1_1, mW1_2, mW2_1, mW2_2, aw2, fW, fb2)

    return Hmat, w.reshape(N)


# speedup vs baseline: 1.1110x; 1.1110x over previous
import functools
import jax
import jax.numpy as jnp
from jax import lax
from jax.experimental import pallas as pl
from jax.experimental.pallas import tpu as pltpu
from jax.experimental.pallas import tpu_sc as plsc

N = 4096
HID = 256
TOP_K = 10
BLK = 512
NBLK = N // BLK
SENT = -1e9

NSUB = 32
CPAN = N // NSUB
RCHUNK = 512
NCHUNK = N // RCHUNK


def _tc_body(x0, x1, x2, w01, w02, w11, w12, w21, w22, aw, fw, fb,
             et_out, keep_out, w_out, nrm_s, nt_s, rsum_s):
    i = pl.program_id(0)

    @pl.when(i < NBLK)
    def _phase_a():
        a = aw[...]
        a = a - jnp.max(a, axis=1, keepdims=True)
        e = jnp.exp(a)
        a = e / jnp.sum(e, axis=1, keepdims=True)

        def mlp(x, w1, w2):
            h = jnp.maximum(jnp.dot(x[...], w1[...], preferred_element_type=jnp.float32), 0.0)
            return jnp.dot(h, w2[...], preferred_element_type=jnp.float32)

        fsum = (a[0, 0] * mlp(x0, w01, w02) + a[0, 1] * mlp(x1, w11, w12)
                + a[0, 2] * mlp(x2, w21, w22))
        fused = jnp.dot(fsum, fw[...], preferred_element_type=jnp.float32) + fb[...]
        nrm = jnp.sqrt(jnp.sum(fused * fused, axis=1, keepdims=True))
        nrm = jnp.maximum(nrm, 1e-12)
        normed = fused / nrm
        nrm_s[pl.ds(i * BLK, BLK), :] = normed
        nt_s[:, pl.ds(i, 1), :] = normed.T.reshape(HID, 1, BLK)
        rsum_s[pl.ds(i * BLK, BLK), :] = jnp.sum(fused, axis=1, keepdims=True)

    @pl.when(i >= NBLK)
    def _phase_b():
        j = i - NBLK
        r0 = j * BLK
        rids_i = r0 + lax.broadcasted_iota(jnp.int32, (BLK, 1), 0)
        rids = rids_i.astype(jnp.float32)
        cols = lax.broadcasted_iota(jnp.int32, (BLK, N), 1).astype(jnp.float32)
        nb = nrm_s[pl.ds(r0, BLK), :]
        nt = nt_s[...].reshape(HID, N)
        sim = jnp.dot(nb, nt, preferred_element_type=jnp.float32)
        sim = jnp.where(cols == rids, SENT, sim)
        picks = [rids]
        for _ in range(TOP_K):
            m = jnp.max(sim, axis=1, keepdims=True)
            cand = jnp.where(sim == m, cols, float(N))
            jj = jnp.min(cand, axis=1, keepdims=True)
            sim = jnp.where(cols == jj, SENT, sim)
            picks.append(jj)
        selmask = (sim == SENT).astype(jnp.float32)
        msum = jnp.dot(selmask, rsum_s[...], preferred_element_type=jnp.float32)
        w = jax.nn.sigmoid(msum / float((TOP_K + 1) * HID))
        keep = w > 0.0
        picks += [jnp.full((BLK, 1), -1.0, jnp.float32)] * (16 - len(picks))
        edges = jnp.concatenate(picks, axis=1).astype(jnp.int32)
        et_out[...] = edges.T
        keep_out[...] = jnp.where(keep, 1.0, 0.0).T.reshape(1, BLK)
        w_out[...] = jnp.where(keep, w, 0.0).T.reshape(1, BLK)


def _sc_body(et_hbm, keep_hbm, zeros_hbm, h_hbm, et_v, keep_v, buf):
    wid = lax.axis_index("s") * 2 + lax.axis_index("c")
    c0 = wid * CPAN
    for j in range(TOP_K + 1):
        pltpu.sync_copy(et_hbm.at[j, pl.ds(c0, CPAN)], et_v.at[pl.ds(j * CPAN, CPAN)])
    pltpu.sync_copy(keep_hbm.at[0, pl.ds(c0, CPAN)], keep_v)
    pltpu.sync_copy(zeros_hbm, buf)

    lanes = lax.iota(jnp.int32, 16)
    zeros16 = jnp.zeros((16,), jnp.float32)

    def chunk(k, _):
        r0 = k * RCHUNK
        for j in range(TOP_K + 1):
            for g in range(CPAN // 16):
                r = et_v[pl.ds(j * CPAN + g * 16, 16)]
                mask = jnp.logical_and(r >= r0, r < r0 + RCHUNK)
                kv = keep_v[pl.ds(g * 16, 16)]
                plsc.store_scatter(buf, [r - r0, lanes + g * 16], kv, mask=mask)
        pltpu.sync_copy(buf, h_hbm.at[pl.ds(r0, RCHUNK), pl.ds(c0, CPAN)])
        for j in range(TOP_K + 1):
            for g in range(CPAN // 16):
                r = et_v[pl.ds(j * CPAN + g * 16, 16)]
                mask = jnp.logical_and(r >= r0, r < r0 + RCHUNK)
                plsc.store_scatter(buf, [r - r0, lanes + g * 16], zeros16, mask=mask)
        return ()

    lax.fori_loop(0, NCHUNK, chunk, ())


def kernel(x0, x1, x2, mW0_1, mb0_1, mW0_2, mb0_2, mW1_1, mb1_1, mW1_2, mb1_2,
           mW2_1, mb2_1, mW2_2, mb2_2, attn_weights, fW, fb):
    f32 = jnp.float32
    aw2 = attn_weights.reshape(1, 3)
    fb2 = fb.reshape(1, HID)

    whole = lambda shape: pl.BlockSpec(shape, lambda i: tuple(0 for _ in shape))
    rows = lambda w: pl.BlockSpec((BLK, w), lambda i: (jnp.minimum(i, NBLK - 1), 0))
    outB = lambda h: pl.BlockSpec((h, BLK), lambda i: (0, jnp.clip(i - NBLK, 0, NBLK - 1)))

    et, keep, w = pl.pallas_call(
        _tc_body,
        grid=(2 * NBLK,),
        in_specs=[
            rows(256), rows(512), rows(128),
            whole((256, HID)), whole((HID, HID)),
            whole((512, HID)), whole((HID, HID)),
            whole((128, HID)), whole((HID, HID)),
            whole((1, 3)), whole((HID, HID)), whole((1, HID)),
        ],
        out_specs=[outB(16), outB(1), outB(1)],
        out_shape=[
            jax.ShapeDtypeStruct((16, N), jnp.int32),
            jax.ShapeDtypeStruct((1, N), f32),
            jax.ShapeDtypeStruct((1, N), f32),
        ],
        scratch_shapes=[
            pltpu.VMEM((N, HID), f32),
            pltpu.VMEM((HID, NBLK, BLK), f32),
            pltpu.VMEM((N, 1), f32),
        ],
    )(x0, x1, x2, mW0_1, mW0_2, mW1_1, mW1_2, mW2_1, mW2_2, aw2, fW, fb2)

    zeros_chunk = jnp.zeros((RCHUNK, CPAN), f32)

    sc_kernel = pl.kernel(
        _sc_body,
        out_type=jax.ShapeDtypeStruct((N, N), f32),
        mesh=plsc.VectorSubcoreMesh(core_axis_name="c", subcore_axis_name="s"),
        compiler_params=pltpu.CompilerParams(needs_layout_passes=False),
        scratch_types=[
            pltpu.VMEM(((TOP_K + 1) * CPAN,), jnp.int32),
            pltpu.VMEM((CPAN,), f32),
            pltpu.VMEM((RCHUNK, CPAN), f32),
        ],
    )
    Hmat = sc_kernel(et, keep, zeros_chunk)

    return Hmat, w.reshape(N)


# speedup vs baseline: 1.1797x; 1.0618x over previous
import jax
import jax.numpy as jnp
from jax import lax
from jax.experimental import pallas as pl
from jax.experimental.pallas import tpu as pltpu

N = 4096
HID = 256
TOP_K = 10
BLK = 256
NBLK = N // BLK
SENT = -1e9


def _body(x0, x1, x2, w01, w02, w11, w12, w21, w22, aw, fw, fb,
          h_out, w_out, nrm_s, nt_s, rsum_s, sel_s):
    i = pl.program_id(0)

    @pl.when(i < NBLK)
    def _phase_a():
        a = aw[...]
        a = a - jnp.max(a, axis=1, keepdims=True)
        e = jnp.exp(a)
        a = e / jnp.sum(e, axis=1, keepdims=True)

        def mlp(x, w1, w2):
            h = jnp.maximum(jnp.dot(x[...], w1[...], preferred_element_type=jnp.float32), 0.0)
            return jnp.dot(h, w2[...], preferred_element_type=jnp.float32)

        fsum = (a[0, 0] * mlp(x0, w01, w02) + a[0, 1] * mlp(x1, w11, w12)
                + a[0, 2] * mlp(x2, w21, w22))
        fused = jnp.dot(fsum, fw[...], preferred_element_type=jnp.float32) + fb[...]
        nrm = jnp.sqrt(jnp.sum(fused * fused, axis=1, keepdims=True))
        nrm = jnp.maximum(nrm, 1e-12)
        normed = fused / nrm
        nrm_s[pl.ds(i * BLK, BLK), :] = normed
        nt_s[:, pl.ds(i, 1), :] = normed.T.reshape(HID, 1, BLK)
        rsum_s[pl.ds(i * BLK, BLK), :] = jnp.sum(fused, axis=1, keepdims=True)

    @pl.when(i >= NBLK)
    def _phase_b():
        j = i - NBLK
        r0 = j * BLK
        rids_i = r0 + lax.broadcasted_iota(jnp.int32, (BLK, 1), 0)
        rids = rids_i.astype(jnp.float32)
        cols = lax.broadcasted_iota(jnp.int32, (BLK, N), 1).astype(jnp.float32)
        nb = nrm_s[pl.ds(r0, BLK), :]
        nt = nt_s[...].reshape(HID, N)
        sim = jnp.dot(nb, nt, preferred_element_type=jnp.float32)
        sim = jnp.where(cols == rids, SENT, sim)

        quota = float(TOP_K)
        simk = sim
        cnt = jnp.zeros((BLK, 1), jnp.float32)
        cntprev = jnp.zeros((BLK, 1), jnp.float32)
        vlast = jnp.full((BLK, 1), SENT, jnp.float32)
        for _ in range(TOP_K):
            m = jnp.max(simk, axis=1, keepdims=True)
            active = cnt < quota
            layer = simk == m
            layerf = jnp.where(layer, 1.0, 0.0)
            c = jnp.sum(layerf, axis=1, keepdims=True)
            simk = jnp.where(jnp.logical_and(layer, active), SENT, simk)
            cntprev = jnp.where(active, cnt, cntprev)
            vlast = jnp.where(active, m, vlast)
            cnt = jnp.where(active, cnt + c, cnt)

        selF = jnp.where(simk == SENT, 1.0, 0.0)
        sel_s[...] = selF

        @pl.when(jnp.any(cnt > quota))
        def _trim():
            s = sel_s[...]
            layerb = jnp.logical_and(s > 0.0, sim == vlast)
            lf = jnp.where(layerb, 1.0, 0.0)
            rank = lf
            k = 1
            while k < N:
                shifted = jnp.concatenate(
                    [jnp.zeros((BLK, k), jnp.float32), rank[:, :N - k]], axis=1)
                rank = rank + shifted
                k *= 2
            drop = jnp.logical_and(layerb, rank > (quota - cntprev))
            sel_s[...] = jnp.where(drop, 0.0, s)

        sel = sel_s[...]
        msum = jnp.dot(sel, rsum_s[...], preferred_element_type=jnp.float32)
        w = jax.nn.sigmoid(msum / float((TOP_K + 1) * HID))
        keep = w > 0.0
        keepf = jnp.where(keep, 1.0, 0.0)
        h_out[...] = (sel * keepf).T
        w_out[...] = jnp.where(keep, w, 0.0).T.reshape(1, BLK)


def kernel(x0, x1, x2, mW0_1, mb0_1, mW0_2, mb0_2, mW1_1, mb1_1, mW1_2, mb1_2,
           mW2_1, mb2_1, mW2_2, mb2_2, attn_weights, fW, fb):
    f32 = jnp.float32
    aw2 = attn_weights.reshape(1, 3)
    fb2 = fb.reshape(1, HID)

    whole = lambda shape: pl.BlockSpec(shape, lambda i: tuple(0 for _ in shape))
    rows = lambda w: pl.BlockSpec((BLK, w), lambda i: (jnp.minimum(i, NBLK - 1), 0))
    outB = lambda h: pl.BlockSpec((h, BLK), lambda i: (0, jnp.clip(i - NBLK, 0, NBLK - 1)))

    Hmat, w = pl.pallas_call(
        _body,
        grid=(2 * NBLK,),
        in_specs=[
            rows(256), rows(512), rows(128),
            whole((256, HID)), whole((HID, HID)),
            whole((512, HID)), whole((HID, HID)),
            whole((128, HID)), whole((HID, HID)),
            whole((1, 3)), whole((HID, HID)), whole((1, HID)),
        ],
        out_specs=[outB(N), outB(1)],
        out_shape=[
            jax.ShapeDtypeStruct((N, N), f32),
            jax.ShapeDtypeStruct((1, N), f32),
        ],
        scratch_shapes=[
            pltpu.VMEM((N, HID), f32),
            pltpu.VMEM((HID, NBLK, BLK), f32),
            pltpu.VMEM((N, 1), f32),
            pltpu.VMEM((BLK, N), f32),
        ],
    )(x0, x1, x2, mW0_1, mW0_2, mW1_1, mW1_2, mW2_1, mW2_2, aw2, fW, fb2)

    return Hmat, w.reshape(N)


# speedup vs baseline: 1.1880x; 1.0070x over previous
import jax
import jax.numpy as jnp
from jax import lax
from jax.experimental import pallas as pl
from jax.experimental.pallas import tpu as pltpu

N = 4096
HID = 256
TOP_K = 10
BLK = 256
NBLK = N // BLK
SENT = -1e9


def _body(x0, x1, x2, w01, w02, w11, w12, w21, w22, aw, fw, fb,
          h_out, w_out, nrm_s, nt_s, rsum_s, sel_s):
    i = pl.program_id(0)

    @pl.when(i < NBLK)
    def _phase_a():
        a = aw[...]
        a = a - jnp.max(a, axis=1, keepdims=True)
        e = jnp.exp(a)
        a = e / jnp.sum(e, axis=1, keepdims=True)

        def mlp(x, w1, w2):
            h = jnp.maximum(jnp.dot(x[...], w1[...], preferred_element_type=jnp.float32), 0.0)
            return jnp.dot(h, w2[...], preferred_element_type=jnp.float32)

        fsum = (a[0, 0] * mlp(x0, w01, w02) + a[0, 1] * mlp(x1, w11, w12)
                + a[0, 2] * mlp(x2, w21, w22))
        fused = jnp.dot(fsum, fw[...], preferred_element_type=jnp.float32) + fb[...]
        nrm = jnp.sqrt(jnp.sum(fused * fused, axis=1, keepdims=True))
        nrm = jnp.maximum(nrm, 1e-12)
        normed = fused / nrm
        nrm_s[pl.ds(i * BLK, BLK), :] = normed
        nt_s[:, pl.ds(i, 1), :] = normed.T.reshape(HID, 1, BLK)
        rsum_s[pl.ds(i * BLK, BLK), :] = jnp.sum(fused, axis=1, keepdims=True)

    @pl.when(i >= NBLK)
    def _phase_b():
        j = i - NBLK
        r0 = j * BLK
        rids_i = r0 + lax.broadcasted_iota(jnp.int32, (BLK, 1), 0)
        rids = rids_i.astype(jnp.float32)
        cols = lax.broadcasted_iota(jnp.int32, (BLK, N), 1).astype(jnp.float32)
        nb = nrm_s[pl.ds(r0, BLK), :]
        nt = nt_s[...].reshape(HID, N)
        sim = jnp.dot(nb, nt, preferred_element_type=jnp.float32)
        sim = jnp.where(cols == rids, SENT, sim)

        quota = float(TOP_K)
        simk = sim
        cnt = jnp.zeros((BLK, 1), jnp.float32)
        cntprev = jnp.zeros((BLK, 1), jnp.float32)
        vlast = jnp.full((BLK, 1), SENT, jnp.float32)
        for _ in range(TOP_K):
            m = jnp.max(simk, axis=1, keepdims=True)
            layer = simk == m
            layerf = jnp.where(layer, 1.0, 0.0)
            c = jnp.sum(layerf, axis=1, keepdims=True)
            simk = jnp.where(layer, SENT, simk)
            active = cnt < quota
            cntprev = jnp.where(active, cnt, cntprev)
            vlast = jnp.where(active, m, vlast)
            cnt = jnp.where(active, cnt + c, cnt)

        selF = jnp.where(jnp.logical_or(sim >= vlast, cols == rids), 1.0, 0.0)
        sel_s[...] = selF

        @pl.when(jnp.any(cnt > quota))
        def _trim():
            s = sel_s[...]
            layerb = jnp.logical_and(s > 0.0, sim == vlast)
            lf = jnp.where(layerb, 1.0, 0.0)
            rank = lf
            k = 1
            while k < N:
                shifted = jnp.concatenate(
                    [jnp.zeros((BLK, k), jnp.float32), rank[:, :N - k]], axis=1)
                rank = rank + shifted
                k *= 2
            drop = jnp.logical_and(layerb, rank > (quota - cntprev))
            sel_s[...] = jnp.where(drop, 0.0, s)

        sel = sel_s[...]
        msum = jnp.dot(sel, rsum_s[...], preferred_element_type=jnp.float32)
        w = jax.nn.sigmoid(msum / float((TOP_K + 1) * HID))
        keep = w > 0.0
        keepf = jnp.where(keep, 1.0, 0.0)
        h_out[...] = (sel * keepf).T
        w_out[...] = jnp.where(keep, w, 0.0).T.reshape(1, BLK)


def kernel(x0, x1, x2, mW0_1, mb0_1, mW0_2, mb0_2, mW1_1, mb1_1, mW1_2, mb1_2,
           mW2_1, mb2_1, mW2_2, mb2_2, attn_weights, fW, fb):
    f32 = jnp.float32
    aw2 = attn_weights.reshape(1, 3)
    fb2 = fb.reshape(1, HID)

    whole = lambda shape: pl.BlockSpec(shape, lambda i: tuple(0 for _ in shape))
    rows = lambda w: pl.BlockSpec((BLK, w), lambda i: (jnp.minimum(i, NBLK - 1), 0))
    outB = lambda h: pl.BlockSpec((h, BLK), lambda i: (0, jnp.clip(i - NBLK, 0, NBLK - 1)))

    Hmat, w = pl.pallas_call(
        _body,
        grid=(2 * NBLK,),
        in_specs=[
            rows(256), rows(512), rows(128),
            whole((256, HID)), whole((HID, HID)),
            whole((512, HID)), whole((HID, HID)),
            whole((128, HID)), whole((HID, HID)),
            whole((1, 3)), whole((HID, HID)), whole((1, HID)),
        ],
        out_specs=[outB(N), outB(1)],
        out_shape=[
            jax.ShapeDtypeStruct((N, N), f32),
            jax.ShapeDtypeStruct((1, N), f32),
        ],
        scratch_shapes=[
            pltpu.VMEM((N, HID), f32),
            pltpu.VMEM((HID, NBLK, BLK), f32),
            pltpu.VMEM((N, 1), f32),
            pltpu.VMEM((BLK, N), f32),
        ],
    )(x0, x1, x2, mW0_1, mW0_2, mW1_1, mW1_2, mW2_1, mW2_2, aw2, fW, fb2)

    return Hmat, w.reshape(N)
